# baseline (device time: 58532 ns/iter reference)
import jax
import jax.numpy as jnp
from jax import lax
from jax.experimental import pallas as pl
from jax.experimental.pallas import tpu as pltpu

N_DEV = 16
LOG2_N = 4
B, Sq, Skv = 2, 128, 128
H_PER = 4
Dh = 64
D_MODEL = 512


def kernel(x, Wq, K_ext, V_ext, Wo):
    i = lax.axis_index("i")
    K_loc = lax.dynamic_slice_in_dim(K_ext, i * H_PER, H_PER, axis=2)
    V_loc = lax.dynamic_slice_in_dim(V_ext, i * H_PER, H_PER, axis=2)

    def body(x_ref, wq_ref, k_ref, v_ref, wo_ref, out_ref,
             recv_ref, send_sems, recv_sems):
        my = lax.axis_index("i")

        qi = lax.broadcasted_iota(jnp.int32, (Sq, Skv), 0) // 64
        kj = lax.broadcasted_iota(jnp.int32, (Sq, Skv), 1) // 64
        mask = (qi == kj) | (kj == 0) | ((qi + kj) % 3 == 0)

        wq = wq_ref[...].astype(jnp.bfloat16)
        wo = wo_ref[...].astype(jnp.bfloat16)
        for b in range(B):
            xb = x_ref[b].astype(jnp.bfloat16)
            q = jnp.dot(xb, wq, preferred_element_type=jnp.float32)
            q = q.reshape(Sq, H_PER, Dh).astype(jnp.bfloat16)
            k = k_ref[b].astype(jnp.bfloat16)
            scores = jnp.einsum(
                "ihd,jhd->hij", q, k, preferred_element_type=jnp.float32
            ) * 0.125
            scores = jnp.where(mask[None, :, :], scores, -1e9)
            s_max = jnp.max(scores, axis=-1, keepdims=True)
            w = jnp.exp(scores - s_max)
            w = w / jnp.sum(w, axis=-1, keepdims=True)
            v = v_ref[b].astype(jnp.bfloat16)
            ctx = jnp.einsum(
                "hij,jhd->ihd", w.astype(jnp.bfloat16), v,
                preferred_element_type=jnp.float32,
            ).reshape(Sq, H_PER * Dh).astype(jnp.bfloat16)
            out_ref[b, :, :] = jnp.dot(
                ctx, wo, preferred_element_type=jnp.float32
            )

        for s in range(LOG2_N):
            partner = my ^ (1 << s)
            rdma = pltpu.make_async_remote_copy(
                src_ref=out_ref,
                dst_ref=recv_ref.at[s],
                send_sem=send_sems.at[s],
                recv_sem=recv_sems.at[s],
                device_id=(partner,),
                device_id_type=pl.DeviceIdType.MESH,
            )
            rdma.start()
            rdma.wait()
            out_ref[...] = out_ref[...] + recv_ref[s]

    return pl.pallas_call(
        body,
        out_shape=jax.ShapeDtypeStruct((B, Sq, D_MODEL), jnp.float32),
        in_specs=[pl.BlockSpec(memory_space=pltpu.VMEM)] * 5,
        out_specs=pl.BlockSpec(memory_space=pltpu.VMEM),
        scratch_shapes=[
            pltpu.VMEM((LOG2_N, B, Sq, D_MODEL), jnp.float32),
            pltpu.SemaphoreType.DMA((LOG2_N,)),
            pltpu.SemaphoreType.DMA((LOG2_N,)),
        ],
    )(x, Wq, K_loc, V_loc, Wo)


# device time: 12732 ns/iter; 4.5972x vs baseline; 4.5972x over previous
import jax
import jax.numpy as jnp
from jax import lax
from jax.experimental import pallas as pl
from jax.experimental.pallas import tpu as pltpu

N_DEV = 16
LOG2_N = 4
B, Sq, Skv = 2, 128, 128
H_PER = 4
Dh = 64
D_MODEL = 512


def kernel(x, Wq, K_ext, V_ext, Wo):
    i = lax.axis_index("i")
    K_loc = lax.dynamic_slice_in_dim(K_ext, i * H_PER, H_PER, axis=2)
    V_loc = lax.dynamic_slice_in_dim(V_ext, i * H_PER, H_PER, axis=2)

    def body(x_ref, wq_ref, k_ref, v_ref, wo_ref, out_ref,
             comm_ref, recv_ref, send_sems, recv_sems):
        my = lax.axis_index("i")

        qi = lax.broadcasted_iota(jnp.int32, (Sq, Skv), 0) // 64
        kj = lax.broadcasted_iota(jnp.int32, (Sq, Skv), 1) // 64
        mask = (qi == kj) | (kj == 0) | ((qi + kj) % 3 == 0)

        wq = wq_ref[...].astype(jnp.bfloat16)
        wo = wo_ref[...].astype(jnp.bfloat16)
        for b in range(B):
            xb = x_ref[b].astype(jnp.bfloat16)
            q = jnp.dot(xb, wq, preferred_element_type=jnp.float32)
            q = q.reshape(Sq, H_PER, Dh).astype(jnp.bfloat16)
            k = k_ref[b].astype(jnp.bfloat16)
            scores = jnp.einsum(
                "ihd,jhd->hij", q, k, preferred_element_type=jnp.float32
            ) * 0.125
            scores = jnp.where(mask[None, :, :], scores, -1e9)
            s_max = jnp.max(scores, axis=-1, keepdims=True)
            w = jnp.exp(scores - s_max)
            w = w / jnp.sum(w, axis=-1, keepdims=True)
            v = v_ref[b].astype(jnp.bfloat16)
            ctx = jnp.einsum(
                "hij,jhd->ihd", w.astype(jnp.bfloat16), v,
                preferred_element_type=jnp.float32,
            ).reshape(Sq, H_PER * Dh).astype(jnp.bfloat16)
            out_ref[b, :, :] = jnp.dot(
                ctx, wo, preferred_element_type=jnp.float32
            )

        for s in range(LOG2_N):
            partner = my ^ (1 << s)
            comm_ref[...] = out_ref[...].astype(jnp.bfloat16)
            rdma = pltpu.make_async_remote_copy(
                src_ref=comm_ref,
                dst_ref=recv_ref.at[s],
                send_sem=send_sems.at[s],
                recv_sem=recv_sems.at[s],
                device_id=(partner,),
                device_id_type=pl.DeviceIdType.MESH,
            )
            rdma.start()
            rdma.wait()
            out_ref[...] = out_ref[...] + recv_ref[s].astype(jnp.float32)

    return pl.pallas_call(
        body,
        out_shape=jax.ShapeDtypeStruct((B, Sq, D_MODEL), jnp.float32),
        in_specs=[pl.BlockSpec(memory_space=pltpu.VMEM)] * 5,
        out_specs=pl.BlockSpec(memory_space=pltpu.VMEM),
        scratch_shapes=[
            pltpu.VMEM((B, Sq, D_MODEL), jnp.bfloat16),
            pltpu.VMEM((LOG2_N, B, Sq, D_MODEL), jnp.bfloat16),
            pltpu.SemaphoreType.DMA((LOG2_N,)),
            pltpu.SemaphoreType.DMA((LOG2_N,)),
        ],
    )(x, Wq, K_loc, V_loc, Wo)
